# Initial kernel scaffold; baseline (speedup 1.0000x reference)
import jax
import jax.numpy as jnp
from jax.experimental import pallas as pl


def kernel(x, pos, table):
    raise NotImplementedError("write your pallas kernel here")


# speedup vs baseline: 1.6770x; 1.6770x over previous
import functools

import jax
import jax.numpy as jnp
from jax.experimental import pallas as pl
from jax.experimental.pallas import tpu as pltpu
from jax.experimental.pallas import tpu_sc as plsc

_EMB = 64
_MAXLEN = 200
_W = 128


def _pos_encoding():
    k = jnp.arange(_MAXLEN, dtype=jnp.float32)[:, None]
    i = jnp.arange(_EMB // 2, dtype=jnp.float32)[None, :]
    denom = jnp.power(10000.0, 2.0 * i / _EMB)
    p = jnp.zeros((_MAXLEN, _EMB), dtype=jnp.float32)
    p = p.at[:, 0::2].set(jnp.sin(k / denom))
    p = p.at[:, 1::2].set(jnp.cos(k / denom))
    return p


def kernel(x, pos, table):
    b, l = x.shape
    n = b * l
    p_table = _pos_encoding()
    xf = x.reshape(1, n).astype(jnp.int32)
    pf = pos.reshape(1, n).astype(jnp.int32)
    mesh = plsc.VectorSubcoreMesh(core_axis_name="c", subcore_axis_name="s")

    @functools.partial(
        pl.kernel,
        out_type=jax.ShapeDtypeStruct((n, _EMB), jnp.float32),
        mesh=mesh,
        scratch_types=[pltpu.VMEM((_W, _EMB), jnp.float32)],
        compiler_params=pltpu.CompilerParams(use_tc_tiling_on_sc=False),
    )
    def sc_kernel(table_hbm, p_hbm, x_hbm, pos_hbm, o_hbm, p_rows):
        def body(xi, pi, o_vmem):
            pltpu.sync_copy(table_hbm.at[xi.at[0]], o_vmem)
            pltpu.sync_copy(p_hbm.at[pi.at[0]], p_rows)

            @pl.loop(0, _W)
            def _(r):
                for c in range(_EMB // 16):
                    sl = pl.ds(c * 16, 16)
                    o_vmem[r, sl] = o_vmem[r, sl] + p_rows[r, sl]

        pltpu.emit_pipeline(
            body,
            grid=(n // _W,),
            in_specs=[
                pl.BlockSpec((1, _W), lambda i: (0, i)),
                pl.BlockSpec((1, _W), lambda i: (0, i)),
            ],
            out_specs=[pl.BlockSpec((_W, _EMB), lambda i: (i, 0))],
            core_axis_name=("c", "s"),
            dimension_semantics=(pltpu.PARALLEL,),
        )(x_hbm, pos_hbm, o_hbm)

    out = sc_kernel(table, p_table, xf, pf)
    return out.reshape(b, l, _EMB)
